# baseline (speedup 1.0000x reference)
import numpy as np
import jax
import jax.numpy as jnp
from jax import lax
from jax.experimental import pallas as pl
from jax.experimental.pallas import tpu as pltpu
from jax.experimental.pallas import tpu_sc as plsc

_INV_SQRT3 = float(1.0 / np.sqrt(3.0))

_B, _C, _D = 160000, 64, 4
_CD = _C * _D
_NC, _NS = 2, 16
_NW = _NC * _NS
_ROWS_W = _B // _NW
_R = 40
_NCHUNK = _ROWS_W // _R


def _sc_body(x_hbm, y_hbm, w_hbm, o_hbm, xb, yb, ob, wb):
    wid = lax.axis_index("s") * _NC + lax.axis_index("c")
    base = wid * _ROWS_W

    pltpu.sync_copy(w_hbm, wb)
    wv = [[wb[pl.ds((j * 4 + g) * 16, 16)] for g in range(4)] for j in range(4)]
    col4 = lax.iota(jnp.int32, 16) * 4

    def chunk_body(k, carry):
        r0 = base + k * _R
        pltpu.sync_copy(x_hbm.at[pl.ds(r0 * _CD, _R * _CD)], xb)
        pltpu.sync_copy(y_hbm.at[pl.ds(r0 * _CD, _R * _CD)], yb)

        def row_body(r, c2):
            for g in range(4):
                cols = col4 + (r * _CD + 64 * g)
                x0 = plsc.load_gather(xb, [cols])
                x1 = plsc.load_gather(xb, [cols + 1])
                x2 = plsc.load_gather(xb, [cols + 2])
                x3 = plsc.load_gather(xb, [cols + 3])
                y0 = plsc.load_gather(yb, [cols])
                y1 = plsc.load_gather(yb, [cols + 1])
                y2 = plsc.load_gather(yb, [cols + 2])
                y3 = plsc.load_gather(yb, [cols + 3])
                res = (wv[0][g] * (x0 * y0)
                       + wv[1][g] * (x0 * (y1 + y2 + y3))
                       + wv[2][g] * ((x1 + x2 + x3) * y0)
                       + wv[3][g] * (x1 * y1 + x2 * y2 + x3 * y3))
                ob[pl.ds(r * _C + 16 * g, 16)] = res
            return c2

        lax.fori_loop(0, _R, row_body, 0)
        pltpu.sync_copy(ob, o_hbm.at[pl.ds(r0 * _C, _R * _C)])
        return carry

    lax.fori_loop(0, _NCHUNK, chunk_body, 0)


def kernel(x, y, weights):
    B, C, D = x.shape
    xf = x.reshape(B * C * D)
    yf = y.reshape(B * C * D)
    wt = jnp.stack(
        [weights[:, 0], weights[:, 1], weights[:, 2],
         weights[:, 3] * _INV_SQRT3], axis=0)
    wtab = wt.reshape(4, 4, 16).reshape(256)

    mesh = plsc.VectorSubcoreMesh(core_axis_name="c", subcore_axis_name="s")
    f = pl.kernel(
        _sc_body,
        out_type=jax.ShapeDtypeStruct((B * C,), jnp.float32),
        mesh=mesh,
        compiler_params=pltpu.CompilerParams(needs_layout_passes=False),
        scratch_types=[
            pltpu.VMEM((_R * _CD,), jnp.float32),
            pltpu.VMEM((_R * _CD,), jnp.float32),
            pltpu.VMEM((_R * _C,), jnp.float32),
            pltpu.VMEM((256,), jnp.float32),
        ],
    )
    return f(xf, yf, wtab).reshape(B, C)


# speedup vs baseline: 1.0007x; 1.0007x over previous
import numpy as np
import jax
import jax.numpy as jnp
from jax import lax
from jax.experimental import pallas as pl
from jax.experimental.pallas import tpu as pltpu
from jax.experimental.pallas import tpu_sc as plsc

_INV_SQRT3 = float(1.0 / np.sqrt(3.0))

_B, _C, _D = 160000, 64, 4
_CD = _C * _D
_NC, _NS = 2, 16
_NW = _NC * _NS
_ROWS_W = _B // _NW
_R = 40
_NCHUNK = _ROWS_W // _R


def _sc_body(x_hbm, y_hbm, w_hbm, o_hbm, xb, yb, ob, wb):
    wid = lax.axis_index("s") * _NC + lax.axis_index("c")
    base = wid * _ROWS_W

    pltpu.sync_copy(w_hbm, wb)
    wv = [[wb[pl.ds((j * 4 + g) * 16, 16)] for g in range(4)] for j in range(4)]
    col4 = lax.iota(jnp.int32, 16) * 4

    def chunk_body(k, carry):
        r0 = base + k * _R
        pltpu.sync_copy(x_hbm.at[pl.ds(r0 * _CD, _R * _CD)], xb)
        pltpu.sync_copy(y_hbm.at[pl.ds(r0 * _CD, _R * _CD)], yb)

        def row_body(r, c2):
            for g in range(4):
                cols = col4 + (r * _CD + 64 * g)
                x0 = plsc.load_gather(xb, [cols])
                x1 = plsc.load_gather(xb, [cols + 1])
                x2 = plsc.load_gather(xb, [cols + 2])
                x3 = plsc.load_gather(xb, [cols + 3])
                y0 = plsc.load_gather(yb, [cols])
                y1 = plsc.load_gather(yb, [cols + 1])
                y2 = plsc.load_gather(yb, [cols + 2])
                y3 = plsc.load_gather(yb, [cols + 3])
                res = (wv[0][g] * (x0 * y0)
                       + wv[1][g] * (x0 * (y1 + y2 + y3))
                       + wv[2][g] * ((x1 + x2 + x3) * y0)
                       + wv[3][g] * (x1 * y1 + x2 * y2 + x3 * y3))
                ob[pl.ds(r * _C + 16 * g, 16)] = res
            return c2

        lax.fori_loop(0, _R, row_body, 0)
        pltpu.sync_copy(ob, o_hbm.at[pl.ds(r0 * _C, _R * _C)])
        return carry

    lax.fori_loop(0, _NCHUNK, chunk_body, 0)


def kernel(x, y, weights):
    B, C, D = x.shape
    xf = x.reshape(B * C * D)
    yf = y.reshape(B * C * D)
    wt = jnp.stack(
        [weights[:, 0], weights[:, 1], weights[:, 2],
         weights[:, 3] * _INV_SQRT3], axis=0)
    wtab = wt.reshape(4, 4, 16).reshape(256)

    mesh = plsc.VectorSubcoreMesh(core_axis_name="c", subcore_axis_name="s")
    f = pl.kernel(
        _sc_body,
        out_type=jax.ShapeDtypeStruct((B * C,), jnp.float32),
        mesh=mesh,
        compiler_params=pltpu.CompilerParams(
            needs_layout_passes=False, use_tc_tiling_on_sc=True),
        scratch_types=[
            pltpu.VMEM((_R * _CD,), jnp.float32),
            pltpu.VMEM((_R * _CD,), jnp.float32),
            pltpu.VMEM((_R * _C,), jnp.float32),
            pltpu.VMEM((256,), jnp.float32),
        ],
    )
    return f(xf, yf, wtab).reshape(B, C)


# speedup vs baseline: 92.8642x; 92.8019x over previous
import numpy as np
import jax
import jax.numpy as jnp
from jax import lax
from jax.experimental import pallas as pl
from jax.experimental.pallas import tpu as pltpu
from jax.experimental.pallas import tpu_sc as plsc

_INV_SQRT3 = float(1.0 / np.sqrt(3.0))

_B, _C, _D = 160000, 64, 4
_NC, _NS = 2, 16
_NW = _NC * _NS
_CPW = _C // _NW
_NB = 6400
_RT = _NB // 128
_NCHUNK = _B // _NB


def _sc_body(x_hbm, y_hbm, w_hbm, o_hbm,
             x0b, x1b, x2b, x3b, y0b, y1b, y2b, y3b, ob, wb):
    wid = lax.axis_index("s") * _NC + lax.axis_index("c")

    pltpu.sync_copy(w_hbm, wb)

    for i in range(_CPW):
        c = wid * _CPW + i
        w0 = wb[pl.ds((c * 4 + 0) * 16, 16)]
        w1 = wb[pl.ds((c * 4 + 1) * 16, 16)]
        w2 = wb[pl.ds((c * 4 + 2) * 16, 16)]
        w3 = wb[pl.ds((c * 4 + 3) * 16, 16)]

        def chunk_body(k, carry, c=c, w0=w0, w1=w1, w2=w2, w3=w3):
            b0 = k * _NB
            pltpu.sync_copy(x_hbm.at[c, 0, pl.ds(b0, _NB)], x0b)
            pltpu.sync_copy(x_hbm.at[c, 1, pl.ds(b0, _NB)], x1b)
            pltpu.sync_copy(x_hbm.at[c, 2, pl.ds(b0, _NB)], x2b)
            pltpu.sync_copy(x_hbm.at[c, 3, pl.ds(b0, _NB)], x3b)
            pltpu.sync_copy(y_hbm.at[c, 0, pl.ds(b0, _NB)], y0b)
            pltpu.sync_copy(y_hbm.at[c, 1, pl.ds(b0, _NB)], y1b)
            pltpu.sync_copy(y_hbm.at[c, 2, pl.ds(b0, _NB)], y2b)
            pltpu.sync_copy(y_hbm.at[c, 3, pl.ds(b0, _NB)], y3b)

            def tg_body(j, c2):
                for q in range(8):
                    s = pl.ds(j * 128 + q * 16, 16)
                    x0 = x0b[s]
                    x1 = x1b[s]
                    x2 = x2b[s]
                    x3 = x3b[s]
                    y0 = y0b[s]
                    y1 = y1b[s]
                    y2 = y2b[s]
                    y3 = y3b[s]
                    res = (w0 * (x0 * y0)
                           + w1 * (x0 * (y1 + y2 + y3))
                           + w2 * ((x1 + x2 + x3) * y0)
                           + w3 * (x1 * y1 + x2 * y2 + x3 * y3))
                    ob[j, pl.ds(q * 16, 16)] = res
                return c2

            lax.fori_loop(0, _RT, tg_body, 0)
            pltpu.sync_copy(
                ob, o_hbm.at[c // 8, pl.ds(k * _RT, _RT), c % 8, :])
            return carry

        lax.fori_loop(0, _NCHUNK, chunk_body, 0)


def kernel(x, y, weights):
    B, C, D = x.shape
    xv = x.transpose(1, 2, 0)
    yv = y.transpose(1, 2, 0)
    wt = jnp.stack(
        [weights[:, 0], weights[:, 1], weights[:, 2],
         weights[:, 3] * _INV_SQRT3], axis=1)
    wsplat = jnp.repeat(wt.reshape(C * 4, 1), 16, axis=1).reshape(C * 4 * 16)

    mesh = plsc.VectorSubcoreMesh(core_axis_name="c", subcore_axis_name="s")
    f = pl.kernel(
        _sc_body,
        out_type=jax.ShapeDtypeStruct((C // 8, B // 128, 8, 128), jnp.float32),
        mesh=mesh,
        compiler_params=pltpu.CompilerParams(needs_layout_passes=False),
        scratch_types=(
            [pltpu.VMEM((_NB,), jnp.float32) for _ in range(8)]
            + [pltpu.VMEM((_RT, 128), jnp.float32),
               pltpu.VMEM((C * 4 * 16,), jnp.float32)]
        ),
    )
    o4 = f(xv, yv, wsplat)
    return o4.transpose(1, 3, 0, 2).reshape(B, C)


# speedup vs baseline: 252.9190x; 2.7235x over previous
import numpy as np
import jax
import jax.numpy as jnp
from jax import lax
from jax.experimental import pallas as pl
from jax.experimental.pallas import tpu as pltpu
from jax.experimental.pallas import tpu_sc as plsc

_INV_SQRT3 = float(1.0 / np.sqrt(3.0))

_B, _C, _D = 160000, 64, 4
_NC, _NS = 2, 16
_NW = _NC * _NS
_CPW = _C // _NW
_NB = 3200
_RT = _NB // 128
_NPAIR = _B // (2 * _NB)


def _compute(xb, yb, ob, w0, w1, w2, w3):
    @plsc.parallel_loop(0, _RT, 1, unroll=2)
    def _(j):
        for q in range(8):
            s = pl.ds(j * 128 + q * 16, 16)
            x0 = xb[0, s]
            x1 = xb[1, s]
            x2 = xb[2, s]
            x3 = xb[3, s]
            y0 = yb[0, s]
            y1 = yb[1, s]
            y2 = yb[2, s]
            y3 = yb[3, s]
            res = (w0 * (x0 * y0)
                   + w1 * (x0 * (y1 + y2 + y3))
                   + w2 * ((x1 + x2 + x3) * y0)
                   + w3 * (x1 * y1 + x2 * y2 + x3 * y3))
            ob[j, pl.ds(q * 16, 16)] = res


def _sc_body(x_hbm, y_hbm, w_hbm, o_hbm,
             xba, yba, xbb, ybb, ob, wb, sema, semb):
    wid = lax.axis_index("s") * _NC + lax.axis_index("c")

    pltpu.sync_copy(w_hbm, wb)

    def in_slice(h, c, chunk):
        return h.at[c, :, pl.ds(chunk * _NB, _NB)]

    for i in range(_CPW):
        c = wid * _CPW + i
        w0 = wb[pl.ds((c * 4 + 0) * 16, 16)]
        w1 = wb[pl.ds((c * 4 + 1) * 16, 16)]
        w2 = wb[pl.ds((c * 4 + 2) * 16, 16)]
        w3 = wb[pl.ds((c * 4 + 3) * 16, 16)]

        pltpu.async_copy(in_slice(x_hbm, c, 0), xba, sema)
        pltpu.async_copy(in_slice(y_hbm, c, 0), yba, sema)

        def pair_body(kk, carry, c=c, w0=w0, w1=w1, w2=w2, w3=w3):
            ka = 2 * kk
            pltpu.async_copy(in_slice(x_hbm, c, ka + 1), xbb, semb)
            pltpu.async_copy(in_slice(y_hbm, c, ka + 1), ybb, semb)

            pltpu.make_async_copy(in_slice(x_hbm, c, ka), xba, sema).wait()
            pltpu.make_async_copy(in_slice(y_hbm, c, ka), yba, sema).wait()
            _compute(xba, yba, ob, w0, w1, w2, w3)
            pltpu.sync_copy(
                ob, o_hbm.at[c // 8, pl.ds(ka * _RT, _RT), c % 8, :])

            @pl.when(kk < _NPAIR - 1)
            def _():
                pltpu.async_copy(in_slice(x_hbm, c, ka + 2), xba, sema)
                pltpu.async_copy(in_slice(y_hbm, c, ka + 2), yba, sema)

            pltpu.make_async_copy(in_slice(x_hbm, c, ka + 1), xbb, semb).wait()
            pltpu.make_async_copy(in_slice(y_hbm, c, ka + 1), ybb, semb).wait()
            _compute(xbb, ybb, ob, w0, w1, w2, w3)
            pltpu.sync_copy(
                ob, o_hbm.at[c // 8, pl.ds((ka + 1) * _RT, _RT), c % 8, :])
            return carry

        lax.fori_loop(0, _NPAIR, pair_body, 0)


def kernel(x, y, weights):
    B, C, D = x.shape
    xv = x.transpose(1, 2, 0)
    yv = y.transpose(1, 2, 0)
    wt = jnp.stack(
        [weights[:, 0], weights[:, 1], weights[:, 2],
         weights[:, 3] * _INV_SQRT3], axis=1)
    wsplat = jnp.repeat(wt.reshape(C * 4, 1), 16, axis=1).reshape(C * 4 * 16)

    mesh = plsc.VectorSubcoreMesh(core_axis_name="c", subcore_axis_name="s")
    f = pl.kernel(
        _sc_body,
        out_type=jax.ShapeDtypeStruct((C // 8, B // 128, 8, 128), jnp.float32),
        mesh=mesh,
        compiler_params=pltpu.CompilerParams(needs_layout_passes=False),
        scratch_types=[
            pltpu.VMEM((_D, _NB), jnp.float32),
            pltpu.VMEM((_D, _NB), jnp.float32),
            pltpu.VMEM((_D, _NB), jnp.float32),
            pltpu.VMEM((_D, _NB), jnp.float32),
            pltpu.VMEM((_RT, 128), jnp.float32),
            pltpu.VMEM((C * 4 * 16,), jnp.float32),
            pltpu.SemaphoreType.DMA,
            pltpu.SemaphoreType.DMA,
        ],
    )
    o4 = f(xv, yv, wsplat)
    return o4.transpose(1, 3, 0, 2).reshape(B, C)


# speedup vs baseline: 254.5251x; 1.0064x over previous
import numpy as np
import jax
import jax.numpy as jnp
from jax import lax
from jax.experimental import pallas as pl
from jax.experimental.pallas import tpu as pltpu
from jax.experimental.pallas import tpu_sc as plsc

_INV_SQRT3 = float(1.0 / np.sqrt(3.0))

_B, _C, _D = 160000, 64, 4
_NC, _NS = 2, 16
_NW = _NC * _NS
_CPW = _C // _NW
_NB = 3200
_RT = _NB // 128
_NPAIR = _B // (2 * _NB)


def _compute(xb, yb, ob, w0, w1, w2, w3):
    @plsc.parallel_loop(0, _RT, 1, unroll=5)
    def _(j):
        for q in range(8):
            s = pl.ds(j * 128 + q * 16, 16)
            x0 = xb[0, s]
            x1 = xb[1, s]
            x2 = xb[2, s]
            x3 = xb[3, s]
            y0 = yb[0, s]
            y1 = yb[1, s]
            y2 = yb[2, s]
            y3 = yb[3, s]
            res = (w0 * (x0 * y0)
                   + w1 * (x0 * (y1 + y2 + y3))
                   + w2 * ((x1 + x2 + x3) * y0)
                   + w3 * (x1 * y1 + x2 * y2 + x3 * y3))
            ob[j, pl.ds(q * 16, 16)] = res


def _sc_body(x_hbm, y_hbm, w_hbm, o_hbm,
             xba, yba, xbb, ybb, oba, obb, wb, sema, semb, semo):
    wid = lax.axis_index("s") * _NC + lax.axis_index("c")

    pltpu.sync_copy(w_hbm, wb)

    def in_slice(h, c, chunk):
        return h.at[c, :, pl.ds(chunk * _NB, _NB)]

    def out_slice(c, chunk):
        return o_hbm.at[c // 8, pl.ds(chunk * _RT, _RT), c % 8, :]

    for i in range(_CPW):
        c = wid * _CPW + i
        w0 = wb[pl.ds((c * 4 + 0) * 16, 16)]
        w1 = wb[pl.ds((c * 4 + 1) * 16, 16)]
        w2 = wb[pl.ds((c * 4 + 2) * 16, 16)]
        w3 = wb[pl.ds((c * 4 + 3) * 16, 16)]

        pltpu.async_copy(in_slice(x_hbm, c, 0), xba, sema)
        pltpu.async_copy(in_slice(y_hbm, c, 0), yba, sema)

        def pair_body(kk, carry, c=c, w0=w0, w1=w1, w2=w2, w3=w3):
            ka = 2 * kk
            pltpu.async_copy(in_slice(x_hbm, c, ka + 1), xbb, semb)
            pltpu.async_copy(in_slice(y_hbm, c, ka + 1), ybb, semb)

            pltpu.make_async_copy(in_slice(x_hbm, c, ka), xba, sema).wait()
            pltpu.make_async_copy(in_slice(y_hbm, c, ka), yba, sema).wait()

            @pl.when(kk > 0)
            def _():
                pltpu.make_async_copy(oba, out_slice(c, 0), semo).wait()

            _compute(xba, yba, oba, w0, w1, w2, w3)
            pltpu.async_copy(oba, out_slice(c, ka), semo)

            @pl.when(kk < _NPAIR - 1)
            def _():
                pltpu.async_copy(in_slice(x_hbm, c, ka + 2), xba, sema)
                pltpu.async_copy(in_slice(y_hbm, c, ka + 2), yba, sema)

            pltpu.make_async_copy(in_slice(x_hbm, c, ka + 1), xbb, semb).wait()
            pltpu.make_async_copy(in_slice(y_hbm, c, ka + 1), ybb, semb).wait()

            @pl.when(kk > 0)
            def _():
                pltpu.make_async_copy(obb, out_slice(c, 0), semo).wait()

            _compute(xbb, ybb, obb, w0, w1, w2, w3)
            pltpu.async_copy(obb, out_slice(c, ka + 1), semo)
            return carry

        lax.fori_loop(0, _NPAIR, pair_body, 0)
        pltpu.make_async_copy(oba, out_slice(c, 0), semo).wait()
        pltpu.make_async_copy(obb, out_slice(c, 0), semo).wait()


def kernel(x, y, weights):
    B, C, D = x.shape
    xv = x.transpose(1, 2, 0)
    yv = y.transpose(1, 2, 0)
    wt = jnp.stack(
        [weights[:, 0], weights[:, 1], weights[:, 2],
         weights[:, 3] * _INV_SQRT3], axis=1)
    wsplat = jnp.repeat(wt.reshape(C * 4, 1), 16, axis=1).reshape(C * 4 * 16)

    mesh = plsc.VectorSubcoreMesh(core_axis_name="c", subcore_axis_name="s")
    f = pl.kernel(
        _sc_body,
        out_type=jax.ShapeDtypeStruct((C // 8, B // 128, 8, 128), jnp.float32),
        mesh=mesh,
        compiler_params=pltpu.CompilerParams(needs_layout_passes=False),
        scratch_types=[
            pltpu.VMEM((_D, _NB), jnp.float32),
            pltpu.VMEM((_D, _NB), jnp.float32),
            pltpu.VMEM((_D, _NB), jnp.float32),
            pltpu.VMEM((_D, _NB), jnp.float32),
            pltpu.VMEM((_RT, 128), jnp.float32),
            pltpu.VMEM((_RT, 128), jnp.float32),
            pltpu.VMEM((C * 4 * 16,), jnp.float32),
            pltpu.SemaphoreType.DMA,
            pltpu.SemaphoreType.DMA,
            pltpu.SemaphoreType.DMA,
        ],
    )
    o4 = f(xv, yv, wsplat)
    return o4.transpose(1, 3, 0, 2).reshape(B, C)


# speedup vs baseline: 258.6481x; 1.0162x over previous
import numpy as np
import jax
import jax.numpy as jnp
from jax import lax
from jax.experimental import pallas as pl
from jax.experimental.pallas import tpu as pltpu
from jax.experimental.pallas import tpu_sc as plsc

_INV_SQRT3 = float(1.0 / np.sqrt(3.0))

_B, _C, _D = 160000, 64, 4
_NC, _NS = 2, 16
_NW = _NC * _NS
_CPW = _C // _NW
_NB = 3200
_RT = _NB // 128
_NPAIR = _B // (2 * _NB)


def _compute(xb, yb, ob, w0, w1, w2, w3):
    @plsc.parallel_loop(0, _RT, 1, unroll=5)
    def _(j):
        for q in range(8):
            s = pl.ds(j * 128 + q * 16, 16)
            x0 = xb[0, s]
            x1 = xb[1, s]
            x2 = xb[2, s]
            x3 = xb[3, s]
            y0 = yb[0, s]
            y1 = yb[1, s]
            y2 = yb[2, s]
            y3 = yb[3, s]
            res = ((w0 * (x0 * y0) + w1 * (x0 * (y1 + y2 + y3)))
                   + (w2 * ((x1 + x2 + x3) * y0)
                      + w3 * ((x1 * y1 + x2 * y2) + x3 * y3)))
            ob[j, pl.ds(q * 16, 16)] = res


def _sc_body(x_hbm, y_hbm, w_hbm, o_hbm,
             xba, yba, xbb, ybb, oba, obb, wb, sema, semb, semo):
    wid = lax.axis_index("s") * _NC + lax.axis_index("c")

    pltpu.sync_copy(w_hbm, wb)

    def in_slice(h, c, chunk):
        return h.at[c, :, pl.ds(chunk * _NB, _NB)]

    def out_slice(c, chunk):
        return o_hbm.at[c // 8, pl.ds(chunk * _RT, _RT), c % 8, :]

    for i in range(_CPW):
        c = wid * _CPW + i
        w0 = wb[pl.ds((c * 4 + 0) * 16, 16)]
        w1 = wb[pl.ds((c * 4 + 1) * 16, 16)]
        w2 = wb[pl.ds((c * 4 + 2) * 16, 16)]
        w3 = wb[pl.ds((c * 4 + 3) * 16, 16)]

        pltpu.async_copy(in_slice(x_hbm, c, 0), xba, sema)
        pltpu.async_copy(in_slice(y_hbm, c, 0), yba, sema)

        def pair_body(kk, carry, c=c, w0=w0, w1=w1, w2=w2, w3=w3):
            ka = 2 * kk
            pltpu.async_copy(in_slice(x_hbm, c, ka + 1), xbb, semb)
            pltpu.async_copy(in_slice(y_hbm, c, ka + 1), ybb, semb)

            pltpu.make_async_copy(in_slice(x_hbm, c, ka), xba, sema).wait()
            pltpu.make_async_copy(in_slice(y_hbm, c, ka), yba, sema).wait()

            @pl.when(kk > 0)
            def _():
                pltpu.make_async_copy(oba, out_slice(c, 0), semo).wait()

            _compute(xba, yba, oba, w0, w1, w2, w3)
            pltpu.async_copy(oba, out_slice(c, ka), semo)

            @pl.when(kk < _NPAIR - 1)
            def _():
                pltpu.async_copy(in_slice(x_hbm, c, ka + 2), xba, sema)
                pltpu.async_copy(in_slice(y_hbm, c, ka + 2), yba, sema)

            pltpu.make_async_copy(in_slice(x_hbm, c, ka + 1), xbb, semb).wait()
            pltpu.make_async_copy(in_slice(y_hbm, c, ka + 1), ybb, semb).wait()

            @pl.when(kk > 0)
            def _():
                pltpu.make_async_copy(obb, out_slice(c, 0), semo).wait()

            _compute(xbb, ybb, obb, w0, w1, w2, w3)
            pltpu.async_copy(obb, out_slice(c, ka + 1), semo)
            return carry

        lax.fori_loop(0, _NPAIR, pair_body, 0)
        pltpu.make_async_copy(oba, out_slice(c, 0), semo).wait()
        pltpu.make_async_copy(obb, out_slice(c, 0), semo).wait()


def kernel(x, y, weights):
    B, C, D = x.shape
    xv = x.transpose(1, 2, 0)
    yv = y.transpose(1, 2, 0)
    wt = jnp.stack(
        [weights[:, 0], weights[:, 1], weights[:, 2],
         weights[:, 3] * _INV_SQRT3], axis=1)
    wsplat = jnp.repeat(wt.reshape(C * 4, 1), 16, axis=1).reshape(C * 4 * 16)

    mesh = plsc.VectorSubcoreMesh(core_axis_name="c", subcore_axis_name="s")
    f = pl.kernel(
        _sc_body,
        out_type=jax.ShapeDtypeStruct((C // 8, B // 128, 8, 128), jnp.float32),
        mesh=mesh,
        compiler_params=pltpu.CompilerParams(needs_layout_passes=False),
        scratch_types=[
            pltpu.VMEM((_D, _NB), jnp.float32),
            pltpu.VMEM((_D, _NB), jnp.float32),
            pltpu.VMEM((_D, _NB), jnp.float32),
            pltpu.VMEM((_D, _NB), jnp.float32),
            pltpu.VMEM((_RT, 128), jnp.float32),
            pltpu.VMEM((_RT, 128), jnp.float32),
            pltpu.VMEM((C * 4 * 16,), jnp.float32),
            pltpu.SemaphoreType.DMA,
            pltpu.SemaphoreType.DMA,
            pltpu.SemaphoreType.DMA,
        ],
    )
    o4 = f(xv, yv, wsplat)
    return o4.transpose(1, 3, 0, 2).reshape(B, C)
